# baseline (speedup 1.0000x reference)
import functools

import jax
import jax.numpy as jnp
from jax import lax
from jax.experimental import pallas as pl
from jax.experimental.pallas import tpu as pltpu
from jax.experimental.pallas import tpu_sc as plsc

NUM_CORES = 2
NUM_SUBCORES = 16
NUM_WORKERS = NUM_CORES * NUM_SUBCORES
LANES = 16
CHUNK = 128
STAGE = 8
NBUF = 2


def _sc_accumulate(feat_pad, src3, dst3, zrows, zdeg, *, n_pad, n_chunks, d):
    rows_per_tile = n_pad // NUM_SUBCORES
    mesh = plsc.VectorSubcoreMesh(core_axis_name="c", subcore_axis_name="s")

    @functools.partial(
        pl.kernel,
        out_type=[
            jax.ShapeDtypeStruct((NUM_CORES, n_pad, d), jnp.float32),
            jax.ShapeDtypeStruct((NUM_WORKERS, n_pad), jnp.float32),
        ],
        mesh=mesh,
        compiler_params=pltpu.CompilerParams(needs_layout_passes=False),
        scratch_types=[
            pltpu.VMEM((STAGE, CHUNK), jnp.int32),
            pltpu.VMEM((STAGE, CHUNK), jnp.int32),
            pltpu.VMEM((NBUF, CHUNK, 128), jnp.float32),
            pltpu.VMEM((n_pad,), jnp.float32),
            pltpu.VMEM_SHARED((n_pad, 128), jnp.float32),
            pltpu.SemaphoreType.DMA,
            pltpu.SemaphoreType.DMA,
            pltpu.SemaphoreType.DMA,
        ],
    )
    def sc_kernel(feat_hbm, src_hbm, dst_hbm, zrows_hbm, zdeg_hbm, agg_out,
                  deg_out, sidx_v, didx_v, rows_v, deg_v, agg_sh, gsem, gsem2,
                  ssem):
        c = lax.axis_index("c")
        s = lax.axis_index("s")
        gw = c * NUM_SUBCORES + s
        base = s * rows_per_tile
        pltpu.sync_copy(zdeg_hbm, deg_v)
        pltpu.sync_copy(zrows_hbm, rows_v.at[0])
        zcopies = [
            pltpu.async_copy(rows_v.at[0], agg_sh.at[pl.ds(base + k, CHUNK)],
                             ssem)
            for k in range(0, rows_per_tile, CHUNK)
        ]
        for zc in zcopies:
            zc.wait()
        plsc.subcore_barrier()

        ones16 = jnp.full((LANES,), 1.0, jnp.float32)

        def group_body(g, carry):
            off = pl.multiple_of(g * STAGE, STAGE)
            pltpu.sync_copy(src_hbm.at[gw, pl.ds(off, STAGE)], sidx_v)
            pltpu.sync_copy(dst_hbm.at[gw, pl.ds(off, STAGE)], didx_v)
            half = CHUNK // 2

            def start_gather(j, b):
                return (
                    pltpu.async_copy(feat_hbm.at[sidx_v.at[j, pl.ds(0, half)]],
                                     rows_v.at[b, pl.ds(0, half)], gsem),
                    pltpu.async_copy(
                        feat_hbm.at[sidx_v.at[j, pl.ds(half, half)]],
                        rows_v.at[b, pl.ds(half, half)], gsem2),
                )

            gathers = [None] * NBUF
            scatters = [None] * NBUF
            for j in range(min(NBUF, STAGE)):
                gathers[j] = start_gather(j, j)
            for j in range(STAGE):
                b = j % NBUF
                gathers[b][0].wait()
                gathers[b][1].wait()
                scatters[b] = pltpu.async_copy(
                    rows_v.at[b], agg_sh.at[didx_v.at[j]], ssem, add=True)
                for i in range(CHUNK // LANES):
                    idx16 = didx_v[j, pl.ds(i * LANES, LANES)]
                    plsc.addupdate_scatter(deg_v, [idx16], ones16)
                if j + NBUF < STAGE:
                    scatters[b].wait()
                    gathers[b] = start_gather(j + NBUF, b)
            for j in range(max(0, STAGE - NBUF), STAGE):
                scatters[j % NBUF].wait()
            return carry

        lax.fori_loop(0, n_chunks // STAGE, group_body, 0)
        plsc.subcore_barrier()
        dsem = gsem
        deg_copy = pltpu.async_copy(deg_v, deg_out.at[gw], ssem)
        n_drain = rows_per_tile // CHUNK
        loads = [None] * n_drain
        stores = [None] * n_drain
        for k in range(min(NBUF, n_drain)):
            loads[k] = pltpu.async_copy(
                agg_sh.at[pl.ds(base + k * CHUNK, CHUNK)], rows_v.at[k], dsem)
        for k in range(n_drain):
            b = k % NBUF
            loads[k].wait()
            stores[k] = pltpu.async_copy(
                rows_v.at[b], agg_out.at[c, pl.ds(base + k * CHUNK, CHUNK)],
                ssem)
            if k + NBUF < n_drain:
                stores[k].wait()
                loads[k + NBUF] = pltpu.async_copy(
                    agg_sh.at[pl.ds(base + (k + NBUF) * CHUNK, CHUNK)],
                    rows_v.at[b], dsem)
        for k in range(max(0, n_drain - NBUF), n_drain):
            stores[k].wait()
        deg_copy.wait()

    return sc_kernel(feat_pad, src3, dst3, zrows, zdeg)


def _tc_finalize_body(agg_ref, deg_ref, feat_ref, w_ref, b_ref, ones_ref,
                      o_ref):
    agg = agg_ref[0] + agg_ref[1]
    deg = lax.dot_general(deg_ref[...], ones_ref[...],
                          dimension_numbers=(((0,), (0,)), ((), ())),
                          preferred_element_type=jnp.float32)
    h = (agg + feat_ref[...]) * lax.rsqrt(jnp.maximum(deg, 1.0))
    o_ref[...] = lax.dot_general(
        h, w_ref[...], dimension_numbers=(((1,), (1,)), ((), ())),
        preferred_element_type=jnp.float32) + b_ref[...]


def _tc_finalize(agg_part, deg_part, features, W, b2, ones32, *, block_rows):
    n, d = features.shape
    grid = (n + block_rows - 1) // block_rows
    return pl.pallas_call(
        _tc_finalize_body,
        grid=(grid,),
        in_specs=[
            pl.BlockSpec((NUM_CORES, block_rows, d), lambda i: (0, i, 0)),
            pl.BlockSpec((NUM_WORKERS, block_rows), lambda i: (0, i)),
            pl.BlockSpec((block_rows, d), lambda i: (i, 0)),
            pl.BlockSpec(W.shape, lambda i: (0, 0)),
            pl.BlockSpec(b2.shape, lambda i: (0, 0)),
            pl.BlockSpec(ones32.shape, lambda i: (0, 0)),
        ],
        out_specs=pl.BlockSpec((block_rows, d), lambda i: (i, 0)),
        out_shape=jax.ShapeDtypeStruct((n, d), jnp.float32),
    )(agg_part, deg_part, features, W, b2, ones32)


def kernel(features, edge_index, W, b):
    n, d = features.shape
    e = edge_index.shape[1]

    align = NUM_SUBCORES * CHUNK
    n_pad = ((n + 1 + align - 1) // align) * align
    per_worker = NUM_WORKERS * CHUNK
    n_chunks = (e + per_worker - 1) // per_worker
    n_chunks = ((n_chunks + STAGE - 1) // STAGE) * STAGE
    e_pad = NUM_WORKERS * CHUNK * n_chunks

    src = jnp.zeros((e_pad,), jnp.int32).at[:e].set(edge_index[0])
    dst = jnp.full((e_pad,), n, jnp.int32).at[:e].set(edge_index[1])
    src3 = src.reshape(NUM_WORKERS, n_chunks, CHUNK)
    dst3 = dst.reshape(NUM_WORKERS, n_chunks, CHUNK)
    zrows = jnp.zeros((CHUNK, d), jnp.float32)
    zdeg = jnp.zeros((n_pad,), jnp.float32)
    ones32 = jnp.ones((NUM_WORKERS, 1), jnp.float32)

    agg_part, deg_part = _sc_accumulate(
        features, src3, dst3, zrows, zdeg, n_pad=n_pad, n_chunks=n_chunks,
        d=d)

    return _tc_finalize(agg_part, deg_part, features, W, b.reshape(1, d),
                        ones32, block_rows=1024)


# speedup vs baseline: 1.3013x; 1.3013x over previous
import functools

import jax
import jax.numpy as jnp
from jax import lax
from jax.experimental import pallas as pl
from jax.experimental.pallas import tpu as pltpu
from jax.experimental.pallas import tpu_sc as plsc

NUM_CORES = 2
NUM_SUBCORES = 16
NUM_WORKERS = NUM_CORES * NUM_SUBCORES
LANES = 16
CHUNK = 128
STAGE = 8
NBUF = 2


def _sc_accumulate(feat_pad, src3, dst3, zrows, zdeg, *, n_pad, n_chunks, d):
    rows_per_tile = n_pad // NUM_SUBCORES
    mesh = plsc.VectorSubcoreMesh(core_axis_name="c", subcore_axis_name="s")

    @functools.partial(
        pl.kernel,
        out_type=[
            jax.ShapeDtypeStruct((NUM_CORES, n_pad, d), jnp.float32),
            jax.ShapeDtypeStruct((NUM_WORKERS, n_pad), jnp.float32),
        ],
        mesh=mesh,
        compiler_params=pltpu.CompilerParams(needs_layout_passes=False),
        scratch_types=[
            pltpu.VMEM((STAGE, CHUNK), jnp.int32),
            pltpu.VMEM((STAGE, CHUNK), jnp.int32),
            pltpu.VMEM((NBUF, CHUNK, 128), jnp.float32),
            pltpu.VMEM((n_pad,), jnp.float32),
            pltpu.VMEM_SHARED((n_pad, 128), jnp.float32),
            pltpu.SemaphoreType.DMA,
            pltpu.SemaphoreType.DMA,
            pltpu.SemaphoreType.DMA,
        ],
    )
    def sc_kernel(feat_hbm, src_hbm, dst_hbm, zrows_hbm, zdeg_hbm, agg_out,
                  deg_out, sidx_v, didx_v, rows_v, deg_v, agg_sh, gsem, gsem2,
                  ssem):
        c = lax.axis_index("c")
        s = lax.axis_index("s")
        gw = c * NUM_SUBCORES + s
        base = s * rows_per_tile
        pltpu.sync_copy(zdeg_hbm, deg_v)
        pltpu.sync_copy(zrows_hbm, rows_v.at[0])
        zcopies = [
            pltpu.async_copy(rows_v.at[0], agg_sh.at[pl.ds(base + k, CHUNK)],
                             ssem)
            for k in range(0, rows_per_tile, CHUNK)
        ]
        for zc in zcopies:
            zc.wait()
        plsc.subcore_barrier()

        ones16 = jnp.full((LANES,), 1.0, jnp.float32)

        def group_body(g, carry):
            off = pl.multiple_of(g * STAGE, STAGE)
            pltpu.sync_copy(src_hbm.at[gw, pl.ds(off, STAGE)], sidx_v)
            pltpu.sync_copy(dst_hbm.at[gw, pl.ds(off, STAGE)], didx_v)
            half = CHUNK // 2

            def start_gather(j, b):
                return (
                    pltpu.async_copy(feat_hbm.at[sidx_v.at[j, pl.ds(0, half)]],
                                     rows_v.at[b, pl.ds(0, half)], gsem),
                    pltpu.async_copy(
                        feat_hbm.at[sidx_v.at[j, pl.ds(half, half)]],
                        rows_v.at[b, pl.ds(half, half)], gsem2),
                )

            gathers = [None] * NBUF
            scatters = [None] * NBUF
            for j in range(min(NBUF, STAGE)):
                gathers[j] = start_gather(j, j)
            for j in range(STAGE):
                b = j % NBUF
                gathers[b][0].wait()
                gathers[b][1].wait()
                scatters[b] = pltpu.async_copy(
                    rows_v.at[b], agg_sh.at[didx_v.at[j]], ssem, add=True)
                for i in range(CHUNK // LANES):
                    idx16 = didx_v[j, pl.ds(i * LANES, LANES)]
                    plsc.addupdate_scatter(deg_v, [idx16], ones16)
                if j + NBUF < STAGE:
                    scatters[b].wait()
                    gathers[b] = start_gather(j + NBUF, b)
            for j in range(max(0, STAGE - NBUF), STAGE):
                scatters[j % NBUF].wait()
            return carry

        lax.fori_loop(0, n_chunks // STAGE, group_body, 0)
        plsc.subcore_barrier()
        dsem = gsem
        deg_copy = pltpu.async_copy(deg_v, deg_out.at[gw], ssem)
        n_drain = rows_per_tile // CHUNK
        loads = [None] * n_drain
        stores = [None] * n_drain
        for k in range(min(NBUF, n_drain)):
            loads[k] = pltpu.async_copy(
                agg_sh.at[pl.ds(base + k * CHUNK, CHUNK)], rows_v.at[k], dsem)
        for k in range(n_drain):
            b = k % NBUF
            loads[k].wait()
            stores[k] = pltpu.async_copy(
                rows_v.at[b], agg_out.at[c, pl.ds(base + k * CHUNK, CHUNK)],
                ssem)
            if k + NBUF < n_drain:
                stores[k].wait()
                loads[k + NBUF] = pltpu.async_copy(
                    agg_sh.at[pl.ds(base + (k + NBUF) * CHUNK, CHUNK)],
                    rows_v.at[b], dsem)
        for k in range(max(0, n_drain - NBUF), n_drain):
            stores[k].wait()
        deg_copy.wait()

    return sc_kernel(feat_pad, src3, dst3, zrows, zdeg)


def _tc_finalize_body(agg_ref, deg_ref, feat_ref, w_ref, b_ref, ones_ref,
                      o_ref):
    agg = agg_ref[0] + agg_ref[1]
    deg = lax.dot_general(deg_ref[...], ones_ref[...],
                          dimension_numbers=(((0,), (0,)), ((), ())),
                          preferred_element_type=jnp.float32)
    h = (agg + feat_ref[...]) * lax.rsqrt(jnp.maximum(deg, 1.0))
    o_ref[...] = lax.dot_general(
        h, w_ref[...], dimension_numbers=(((1,), (1,)), ((), ())),
        preferred_element_type=jnp.float32) + b_ref[...]


def _tc_finalize(agg_part, deg_part, features, W, b2, ones32, *, block_rows):
    n, d = features.shape
    grid = n // block_rows
    return pl.pallas_call(
        _tc_finalize_body,
        grid=(grid,),
        in_specs=[
            pl.BlockSpec((NUM_CORES, block_rows, d), lambda i: (0, i, 0)),
            pl.BlockSpec((NUM_WORKERS, block_rows), lambda i: (0, i)),
            pl.BlockSpec((block_rows, d), lambda i: (i, 0)),
            pl.BlockSpec(W.shape, lambda i: (0, 0)),
            pl.BlockSpec(b2.shape, lambda i: (0, 0)),
            pl.BlockSpec(ones32.shape, lambda i: (0, 0)),
        ],
        out_specs=pl.BlockSpec((block_rows, d), lambda i: (i, 0)),
        out_shape=jax.ShapeDtypeStruct((n, d), jnp.float32),
    )(agg_part, deg_part, features, W, b2, ones32)


def kernel(features, edge_index, W, b):
    n, d = features.shape
    e = edge_index.shape[1]

    align = NUM_SUBCORES * CHUNK
    n_pad = ((n + 1 + align - 1) // align) * align
    per_worker = NUM_WORKERS * CHUNK
    n_chunks = (e + per_worker - 1) // per_worker
    n_chunks = ((n_chunks + STAGE - 1) // STAGE) * STAGE
    e_pad = NUM_WORKERS * CHUNK * n_chunks

    src = jnp.full((e_pad,), n, jnp.int32).at[:e].set(edge_index[0])
    dst = jnp.full((e_pad,), n, jnp.int32).at[:e].set(edge_index[1])
    src3 = src.reshape(NUM_WORKERS, n_chunks, CHUNK)
    dst3 = dst.reshape(NUM_WORKERS, n_chunks, CHUNK)
    feat_pad = jnp.zeros((n_pad, d), jnp.float32).at[:n].set(features)
    zrows = jnp.zeros((CHUNK, d), jnp.float32)
    zdeg = jnp.zeros((n_pad,), jnp.float32)
    ones32 = jnp.ones((NUM_WORKERS, 1), jnp.float32)

    agg_part, deg_part = _sc_accumulate(
        feat_pad, src3, dst3, zrows, zdeg, n_pad=n_pad, n_chunks=n_chunks,
        d=d)

    out_pad = _tc_finalize(agg_part, deg_part, feat_pad, W, b.reshape(1, d),
                           ones32, block_rows=1024)
    return out_pad[:n]
